# baseline (speedup 1.0000x reference)
import functools

import jax
import jax.numpy as jnp
from jax import lax
from jax.experimental import pallas as pl
from jax.experimental.pallas import tpu as pltpu
from jax.experimental.pallas import tpu_sc as plsc

_B, _T, _D = 4, 8192, 1024
_NC, _NS = 2, 16
_NW = _NC * _NS
_ROWS = _T // _NW
_PLAN = [(0, 64, 0), (64, 48, 1), (112, 64, 0), (176, 48, 1), (224, 32, 0)]
_SLAB = _T // _NC
_SCH = 64
_NSCH = _SLAB // _SCH

_mesh = plsc.VectorSubcoreMesh(core_axis_name="c", subcore_axis_name="s")


@functools.partial(
    pl.kernel,
    mesh=_mesh,
    out_type=jax.ShapeDtypeStruct((_B, _T, _D), jnp.float32),
    scratch_types=[
        pltpu.VMEM((64, _D), jnp.float32),
        pltpu.VMEM((48, _D), jnp.float32),
        pltpu.VMEM_SHARED((2, _SCH, _D), jnp.float32),
        pltpu.SemaphoreType.DMA,
        pltpu.SemaphoreType.DMA,
        pltpu.SemaphoreType.DMA,
        pltpu.SemaphoreType.DMA,
        pltpu.SemaphoreType.DMA,
        pltpu.SemaphoreType.DMA,
        pltpu.SemaphoreType.DMA,
        pltpu.SemaphoreType.DMA,
    ],
)
def _pe_broadcast(pe_hbm, out_hbm, buf0, buf1, spmem, rsem0, rsem1, wsem0,
                  wsem1, srsem0, srsem1, swsem0, swsem1):
    cid = lax.axis_index("c")
    sid = lax.axis_index("s")
    base = cid * _SLAB + sid * _ROWS
    bufs = (buf0, buf1)
    rsems = (rsem0, rsem1)
    wsems = (wsem0, wsem1)

    @pl.when(sid == 0)
    def _spmem_path():
        sbase = cid * _SLAB
        srsems = (srsem0, srsem1)
        swsems = (swsem0, swsem1)
        sreads = [None, None]
        swrites = [None, None]
        sreads[0] = pltpu.async_copy(
            pe_hbm.at[pl.ds(sbase, _SCH)], spmem.at[0], srsem0)
        for k in range(_NSCH):
            i = k % 2
            j = (k + 1) % 2
            sreads[i].wait()
            if k + 1 < _NSCH:
                if swrites[j] is not None:
                    swrites[j].wait()
                sreads[j] = pltpu.async_copy(
                    pe_hbm.at[pl.ds(sbase + (k + 1) * _SCH, _SCH)],
                    spmem.at[j], srsems[j])
            swrites[i] = pltpu.async_copy(
                spmem.at[i], out_hbm.at[_B - 1, pl.ds(sbase + k * _SCH, _SCH)],
                swsems[i])
        for w in swrites:
            if w is not None:
                w.wait()

    def start_read(off, size, s):
        return pltpu.async_copy(pe_hbm.at[pl.ds(base + off, size)],
                                bufs[s].at[pl.ds(0, size)], rsems[s])

    reads = [None, None]
    writes = [None, None]
    reads[_PLAN[0][2]] = start_read(*_PLAN[0])
    for c, (off, size, s) in enumerate(_PLAN):
        reads[s].wait()
        writes_c = [
            pltpu.async_copy(bufs[s].at[pl.ds(0, size)],
                             out_hbm.at[b, pl.ds(base + off, size)], wsems[s])
            for b in range(_B - 1)
        ]
        if c + 1 < len(_PLAN):
            s2 = _PLAN[c + 1][2]
            if writes[s2] is not None:
                for w in writes[s2]:
                    w.wait()
                writes[s2] = None
            reads[s2] = start_read(*_PLAN[c + 1])
        writes[s] = writes_c
    for ws in writes:
        if ws is not None:
            for w in ws:
                w.wait()


def kernel(x, pe_weight):
    del x
    return _pe_broadcast(pe_weight)


# speedup vs baseline: 2.0089x; 2.0089x over previous
import functools

import jax
import jax.numpy as jnp
from jax import lax
from jax.experimental import pallas as pl
from jax.experimental.pallas import tpu as pltpu
from jax.experimental.pallas import tpu_sc as plsc

_B, _T, _D = 4, 8192, 1024
_NC, _NS = 2, 16
_NW = _NC * _NS
_ROWS = _T // _NW
_PLAN = [(0, 64, 0), (64, 48, 1), (112, 64, 0), (176, 48, 1), (224, 32, 0)]

_mesh = plsc.VectorSubcoreMesh(core_axis_name="c", subcore_axis_name="s")


@functools.partial(
    pl.kernel,
    mesh=_mesh,
    out_type=jax.ShapeDtypeStruct((_B, _T, _D), jnp.float32),
    scratch_types=[
        pltpu.VMEM((64, _D), jnp.float32),
        pltpu.VMEM((48, _D), jnp.float32),
        pltpu.SemaphoreType.DMA,
        pltpu.SemaphoreType.DMA,
        pltpu.SemaphoreType.DMA,
        pltpu.SemaphoreType.DMA,
    ],
)
def _pe_broadcast(pe_hbm, out_hbm, buf0, buf1, rsem0, rsem1, wsem0, wsem1):
    wid = lax.axis_index("s") * _NC + lax.axis_index("c")
    base = wid * _ROWS
    bufs = (buf0, buf1)
    rsems = (rsem0, rsem1)
    wsems = (wsem0, wsem1)

    def start_read(off, size, s):
        return pltpu.async_copy(pe_hbm.at[pl.ds(base + off, size)],
                                bufs[s].at[pl.ds(0, size)], rsems[s])

    reads = [None, None]
    writes = [None, None]
    reads[_PLAN[0][2]] = start_read(*_PLAN[0])
    for c, (off, size, s) in enumerate(_PLAN):
        reads[s].wait()
        writes_c = [
            pltpu.async_copy(bufs[s].at[pl.ds(0, size)],
                             out_hbm.at[b, pl.ds(base + off, size)], wsems[s])
            for b in range(_B)
        ]
        if c + 1 < len(_PLAN):
            s2 = _PLAN[c + 1][2]
            if writes[s2] is not None:
                for w in writes[s2]:
                    w.wait()
                writes[s2] = None
            reads[s2] = start_read(*_PLAN[c + 1])
        writes[s] = writes_c
    for ws in writes:
        if ws is not None:
            for w in ws:
                w.wait()


def kernel(x, pe_weight):
    del x
    return _pe_broadcast(pe_weight)
